# baseline (speedup 1.0000x reference)
import functools

import jax
import jax.numpy as jnp
from jax import lax
from jax.experimental import pallas as pl
from jax.experimental.pallas import tpu as pltpu
from jax.experimental.pallas import tpu_sc as plsc

B = 16384
NC = 2
NS = 16
NW = NC * NS
BPW = B // NW
L = 16

NUSERS_PAD = 1000448
NQUESTIONS_PAD = 100352
QTAB_LEN = 2 * NQUESTIONS_PAD


def _tec_body(uid_hbm, qid_hbm, theta_hbm, qtab_hbm, out_hbm,
              uid_v, qid_v, th_v, be_v, al_v, out_v, sem_u, sem_q, sem):
    wid = lax.axis_index("s") * NC + lax.axis_index("c")
    base = wid * BPW

    cu = pltpu.async_copy(uid_hbm.at[pl.ds(base, BPW)], uid_v, sem_u)
    cq = pltpu.async_copy(qid_hbm.at[pl.ds(base, BPW)], qid_v, sem_q)

    H = BPW // 2
    alpha_view = qtab_hbm.at[pl.ds(NQUESTIONS_PAD, NQUESTIONS_PAD)]
    beta_view = qtab_hbm.at[pl.ds(0, NQUESTIONS_PAD)]
    cq.wait()
    cu.wait()
    gath = []
    for h in range(2):
        s = pl.ds(h * H, H)
        gath.append((
            pltpu.async_copy(alpha_view.at[qid_v.at[s]], al_v.at[s], sem_q),
            pltpu.async_copy(beta_view.at[qid_v.at[s]], be_v.at[s], sem),
            pltpu.async_copy(theta_hbm.at[uid_v.at[s]], th_v.at[s], sem),
        ))
    outc = []
    for h in range(2):
        for c in gath[h]:
            c.wait()
        for i in range(h * (H // L), (h + 1) * (H // L)):
            s = pl.ds(i * L, L)
            a = al_v[s]
            v = jnp.exp(-jnp.abs(a))
            z = v / (v + 2.0)
            z2 = z * z
            p = 1.0 + z2 * (1.0 / 3.0 + z2 * (1.0 / 5.0))
            sp = jnp.maximum(a, 0.0) + 2.0 * z * p
            d = sp * (th_v[s] - be_v[s])
            out_v[s] = 1.0 / (1.0 + jnp.exp(-d))
        s = pl.ds(h * H, H)
        outc.append(pltpu.async_copy(out_v.at[s], out_hbm.at[pl.ds(base + h * H, H)], sem_u))
    for c in outc:
        c.wait()


@functools.partial(
    pl.kernel,
    out_type=jax.ShapeDtypeStruct((B,), jnp.float32),
    mesh=plsc.VectorSubcoreMesh(core_axis_name="c", subcore_axis_name="s"),
    scratch_types=[
        pltpu.VMEM((BPW,), jnp.int32),
        pltpu.VMEM((BPW,), jnp.int32),
        pltpu.VMEM((BPW,), jnp.float32),
        pltpu.VMEM((BPW,), jnp.float32),
        pltpu.VMEM((BPW,), jnp.float32),
        pltpu.VMEM((BPW,), jnp.float32),
        pltpu.SemaphoreType.DMA,
        pltpu.SemaphoreType.DMA,
        pltpu.SemaphoreType.DMA,
    ],
)
def _irt_sc_kernel(uid, qid, theta, qtab, out,
                   uid_v, qid_v, th_v, be_v, al_v, out_v, sem_u, sem_q, sem):
    _tec_body(uid, qid, theta, qtab, out,
              uid_v, qid_v, th_v, be_v, al_v, out_v, sem_u, sem_q, sem)


def _linearize(table, n_pad):
    n = table.shape[0]
    return jnp.pad(table.reshape(1, n), ((0, 0), (0, n_pad - n))).reshape(n_pad)


def kernel(user_id, question_id, theta, beta, alpha):
    uid = user_id.astype(jnp.int32)
    qid = question_id.astype(jnp.int32)
    n = beta.shape[0]
    qtab = jnp.pad(
        jnp.concatenate([beta.reshape(1, n), alpha.reshape(1, n)], axis=0),
        ((0, 0), (0, NQUESTIONS_PAD - n))).reshape(QTAB_LEN)
    out = _irt_sc_kernel(uid, qid, _linearize(theta, NUSERS_PAD), qtab)
    return out.reshape(B, 1)


# speedup vs baseline: 1.0427x; 1.0427x over previous
import functools

import jax
import jax.numpy as jnp
from jax import lax
from jax.experimental import pallas as pl
from jax.experimental.pallas import tpu as pltpu
from jax.experimental.pallas import tpu_sc as plsc

B = 16384
NC = 2
NS = 16
NW = NC * NS
BPW = B // NW
L = 16

NUSERS_PAD = 1000448
NQUESTIONS_PAD = 100352
QTAB_LEN = 2 * NQUESTIONS_PAD


def _tec_body(uid_hbm, qid_hbm, theta_hbm, qtab_hbm, out_hbm,
              uid_v, qid_v, th_v, be_v, al_v, out_v, sem_u, sem_q, sem):
    wid = lax.axis_index("s") * NC + lax.axis_index("c")
    base = wid * BPW

    cu = pltpu.async_copy(uid_hbm.at[pl.ds(base, BPW)], uid_v, sem_u)
    cq = pltpu.async_copy(qid_hbm.at[pl.ds(base, BPW)], qid_v, sem_q)

    cq.wait()
    ca = pltpu.async_copy(
        qtab_hbm.at[pl.ds(NQUESTIONS_PAD, NQUESTIONS_PAD)].at[qid_v], al_v, sem_q)
    cb = pltpu.async_copy(qtab_hbm.at[pl.ds(0, NQUESTIONS_PAD)].at[qid_v], be_v, sem)
    cu.wait()
    ct = pltpu.async_copy(theta_hbm.at[uid_v], th_v, sem)
    ca.wait()

    for i in range(BPW // L):
        s = pl.ds(i * L, L)
        a = al_v[s]
        v = jnp.exp(-jnp.abs(a))
        z = v / (v + 2.0)
        z2 = z * z
        p = 1.0 + z2 * (1.0 / 3.0 + z2 * (1.0 / 5.0))
        al_v[s] = jnp.maximum(a, 0.0) + 2.0 * z * p

    cb.wait()
    ct.wait()
    for i in range(BPW // L):
        s = pl.ds(i * L, L)
        d = al_v[s] * (th_v[s] - be_v[s])
        out_v[s] = 1.0 / (1.0 + jnp.exp(-d))

    pltpu.sync_copy(out_v, out_hbm.at[pl.ds(base, BPW)])


@functools.partial(
    pl.kernel,
    out_type=jax.ShapeDtypeStruct((B,), jnp.float32),
    mesh=plsc.VectorSubcoreMesh(core_axis_name="c", subcore_axis_name="s"),
    scratch_types=[
        pltpu.VMEM((BPW,), jnp.int32),
        pltpu.VMEM((BPW,), jnp.int32),
        pltpu.VMEM((BPW,), jnp.float32),
        pltpu.VMEM((BPW,), jnp.float32),
        pltpu.VMEM((BPW,), jnp.float32),
        pltpu.VMEM((BPW,), jnp.float32),
        pltpu.SemaphoreType.DMA,
        pltpu.SemaphoreType.DMA,
        pltpu.SemaphoreType.DMA,
    ],
)
def _irt_sc_kernel(uid, qid, theta, qtab, out,
                   uid_v, qid_v, th_v, be_v, al_v, out_v, sem_u, sem_q, sem):
    _tec_body(uid, qid, theta, qtab, out,
              uid_v, qid_v, th_v, be_v, al_v, out_v, sem_u, sem_q, sem)


def _linearize(table, n_pad):
    n = table.shape[0]
    return jnp.pad(table.reshape(1, n), ((0, 0), (0, n_pad - n))).reshape(n_pad)


def kernel(user_id, question_id, theta, beta, alpha):
    uid = user_id.astype(jnp.int32)
    qid = question_id.astype(jnp.int32)
    n = beta.shape[0]
    qtab = jnp.pad(
        jnp.concatenate([beta.reshape(1, n), alpha.reshape(1, n)], axis=0),
        ((0, 0), (0, NQUESTIONS_PAD - n))).reshape(QTAB_LEN)
    out = _irt_sc_kernel(uid, qid, _linearize(theta, NUSERS_PAD), qtab)
    return out.reshape(B, 1)
